# baseline (speedup 1.0000x reference)
import functools

import jax
import jax.numpy as jnp
from jax import lax
from jax.experimental import pallas as pl
from jax.experimental.pallas import tpu as pltpu
from jax.experimental.pallas import tpu_sc as plsc

N = 10000
E = 160000
D_IN = 256
H = 512
D_OUT = 256
D_EDGE = 16
L = 3

_ROW_BLK = 2000
_EDGE_BLK = 4000
_CW = 128
_NCHUNK = H // _CW

_NS = 16
_EPT = E // _NS
_EB = 80
_NBLK = _EPT // _EB
_NH = 5112
_TRASH = _NH
_ACC_ROWS = 5120
_NPAD = 2 * _NH
_STRIPE = _ACC_ROWS // _NS


def _prep_h_body(h_ref, w_ref, b_ref, o0, o1, o2, o3):
    y = h_ref[...] @ w_ref[...] + b_ref[...]
    for c, o in enumerate((o0, o1, o2, o3)):
        o[...] = y[:, c * _CW:(c + 1) * _CW]


def _prep_h(h, W_in, b_in):
    cspec = pl.BlockSpec((_ROW_BLK, _CW), lambda i: (i, 0))
    return pl.pallas_call(
        _prep_h_body,
        grid=(N // _ROW_BLK,),
        in_specs=[
            pl.BlockSpec((_ROW_BLK, D_IN), lambda i: (i, 0)),
            pl.BlockSpec((D_IN, H), lambda i: (0, 0)),
            pl.BlockSpec((1, H), lambda i: (0, 0)),
        ],
        out_specs=[cspec] * _NCHUNK,
        out_shape=[jax.ShapeDtypeStruct((N, _CW), jnp.float32)] * _NCHUNK,
    )(h, W_in, b_in.reshape(1, H))


def _prep_ea_body(ea_ref, w_ref, b_ref, o0, o1, o2, o3):
    y = ea_ref[...] @ w_ref[...] + b_ref[...]
    for c, o in enumerate((o0, o1, o2, o3)):
        o[...] = y[:, c * _CW:(c + 1) * _CW]


def _prep_ea(edge_attr, W_e, b_e):
    cspec = pl.BlockSpec((_EDGE_BLK, _CW), lambda i: (i, 0))
    return pl.pallas_call(
        _prep_ea_body,
        grid=(E // _EDGE_BLK,),
        in_specs=[
            pl.BlockSpec((_EDGE_BLK, D_EDGE), lambda i: (i, 0)),
            pl.BlockSpec((D_EDGE, H), lambda i: (0, 0)),
            pl.BlockSpec((1, H), lambda i: (0, 0)),
        ],
        out_specs=[cspec] * _NCHUNK,
        out_shape=[jax.ShapeDtypeStruct((E, _CW), jnp.float32)] * _NCHUNK,
    )(edge_attr, W_e, b_e.reshape(1, H))


def _mlp_body(h0, h1, h2, h3, a0, a1, a2, a3, w0_ref, b0_ref, w1_ref, b1_ref,
              w2_ref, b2_ref, wo_ref, bo_ref, o0, o1, o2, o3):
    x = jnp.concatenate(
        [hc[...] + ac[...] for hc, ac in zip((h0, h1, h2, h3),
                                             (a0, a1, a2, a3))], axis=1)
    x = jnp.maximum(x @ w0_ref[...] + b0_ref[...], 0.0)
    x = jnp.maximum(x @ w1_ref[...] + b1_ref[...], 0.0)
    x = jnp.maximum(x @ w2_ref[...] + b2_ref[...], 0.0)
    x = jnp.maximum(x @ wo_ref[...] + bo_ref[...], 0.0)
    for c, o in enumerate((o0, o1, o2, o3)):
        o[...] = x[:, c * _CW:(c + 1) * _CW]


def _mlp(hs, aggrs, W0, b0, W1, b1, W2, b2, Wo, bo):
    wspec = pl.BlockSpec((H, H), lambda i: (0, 0))
    bspec = pl.BlockSpec((1, H), lambda i: (0, 0))
    cspec = pl.BlockSpec((_ROW_BLK, _CW), lambda i: (i, 0))
    return pl.pallas_call(
        _mlp_body,
        grid=(N // _ROW_BLK,),
        in_specs=[cspec] * 8 + [wspec, bspec] * 4,
        out_specs=[cspec] * _NCHUNK,
        out_shape=[jax.ShapeDtypeStruct((N, _CW), jnp.float32)] * _NCHUNK,
    )(*hs, *aggrs, W0, b0.reshape(1, H), W1, b1.reshape(1, H), W2,
      b2.reshape(1, H), Wo, bo.reshape(1, H))


def _proj_out_body(h0, h1, h2, h3, w_ref, b_ref, o_ref):
    x = jnp.concatenate([hc[...] for hc in (h0, h1, h2, h3)], axis=1)
    o_ref[...] = x @ w_ref[...] + b_ref[...]


def _proj_out(hs, W_out, b_out):
    cspec = pl.BlockSpec((_ROW_BLK, _CW), lambda i: (i, 0))
    return pl.pallas_call(
        _proj_out_body,
        grid=(N // _ROW_BLK,),
        in_specs=[cspec] * _NCHUNK + [
            pl.BlockSpec((H, D_OUT), lambda i: (0, 0)),
            pl.BlockSpec((1, D_OUT), lambda i: (0, 0)),
        ],
        out_specs=pl.BlockSpec((_ROW_BLK, D_OUT), lambda i: (i, 0)),
        out_shape=jax.ShapeDtypeStruct((N, D_OUT), jnp.float32),
    )(*hs, W_out, b_out.reshape(1, D_OUT))


def _edge_body(ht0, ht1, ht2, ht3, ea0, ea1, ea2, ea3, srcs, dsts,
               o0, o1, o2, o3, idx_src, idx_dst, adj_stage, gbuf, ebuf, sbuf,
               accum, gsem, esem):
    sid = lax.axis_index("s")
    cid = lax.axis_index("c")
    base_row = sid * _STRIPE

    pltpu.sync_copy(srcs.at[sid], idx_src)
    pltpu.sync_copy(dsts.at[sid], idx_dst)

    zvec = jnp.zeros((16,), jnp.float32)

    def _zero_accum():
        gb = gbuf.at[0]

        def zrow(r, carry):
            for k in range(_CW // 16):
                gb[r, pl.ds(k * 16, 16)] = zvec
            return carry

        lax.fori_loop(0, _EB, zrow, 0, unroll=4)
        for i in range(_STRIPE // _EB):
            pltpu.sync_copy(gb, accum.at[pl.ds(base_row + i * _EB, _EB)])

    _trash_v = _TRASH + (lax.iota(jnp.int32, 16) & 7)

    def _make_adj(b, half):
        lo = half * _NH
        for k in range(_EB // 16):
            adj = idx_src[b, pl.ds(k * 16, 16)] - lo
            ok = (adj >= 0) & (adj < _NH)
            adj_stage[0, pl.ds(k * 16, 16)] = jnp.where(ok, adj, _trash_v)

    def _gather_descr(b, s, ht, ea):
        g = pltpu.make_async_copy(ht.at[idx_dst.at[b]], gbuf.at[s],
                                  gsem.at[s])
        e = pltpu.make_async_copy(
            ea.at[pl.ds(sid * _EPT + b * _EB, _EB)], ebuf.at[s], esem.at[s])
        return g, e

    def _compute(s):
        gb = gbuf.at[s]
        eb = ebuf.at[s]

        def row(r, carry):
            for k in range(_CW // 16):
                ix = (r, pl.ds(k * 16, 16))
                sbuf[ix] = jnp.maximum(gb[ix] + eb[ix], 0.0)
            return carry

        lax.fori_loop(0, _EB, row, 0, unroll=4)

    def _run_pass(ht, ea, out, half):
        _zero_accum()
        plsc.subcore_barrier()

        for s in (0, 1):
            g, e = _gather_descr(s, s, ht, ea)
            g.start()
            e.start()

        def _do_block(b, s, start_next):
            g, e = _gather_descr(b, s, ht, ea)
            g.wait()
            e.wait()
            _compute(s)
            _make_adj(b, half)
            pltpu.sync_copy(sbuf, accum.at[adj_stage.at[0]], add=True)
            if start_next:
                @pl.when(b + 2 < _NBLK)
                def _():
                    gn, en = _gather_descr(b + 2, s, ht, ea)
                    gn.start()
                    en.start()

        def step(gi, carry):
            for s in (0, 1):
                _do_block(2 * gi + s, s, True)
            return carry

        lax.fori_loop(0, _NBLK // 2, step, 0)
        if _NBLK % 2:
            _do_block(_NBLK - 1, (_NBLK - 1) % 2, False)
        plsc.subcore_barrier()

        @pl.when(sid < _NS - 1)
        def _():
            pltpu.sync_copy(
                accum.at[pl.ds(base_row, _STRIPE)],
                out.at[pl.ds(half * _NH + base_row, _STRIPE)])

        @pl.when(sid == _NS - 1)
        def _():
            pltpu.sync_copy(
                accum.at[pl.ds(base_row, _NH - (_NS - 1) * _STRIPE)],
                out.at[pl.ds(half * _NH + base_row,
                             _NH - (_NS - 1) * _STRIPE)])

        plsc.subcore_barrier()

    @pl.when(cid == 0)
    def _():
        _run_pass(ht0, ea0, o0, 0)
        _run_pass(ht0, ea0, o0, 1)
        _run_pass(ht1, ea1, o1, 0)
        _run_pass(ht1, ea1, o1, 1)

    @pl.when(cid == 1)
    def _():
        _run_pass(ht2, ea2, o2, 0)
        _run_pass(ht2, ea2, o2, 1)
        _run_pass(ht3, ea3, o3, 0)
        _run_pass(ht3, ea3, o3, 1)


def _edge_pass(hs, eas, srcs, dsts):
    f = pl.kernel(
        _edge_body,
        out_type=[jax.ShapeDtypeStruct((_NPAD, _CW), jnp.float32)] * _NCHUNK,
        mesh=plsc.VectorSubcoreMesh(core_axis_name="c", subcore_axis_name="s"),
        scratch_types=[
            pltpu.VMEM((_NBLK, _EB), jnp.int32),
            pltpu.VMEM((_NBLK, _EB), jnp.int32),
            pltpu.VMEM((8, _EB), jnp.int32),
            pltpu.VMEM((2, _EB, _CW), jnp.float32),
            pltpu.VMEM((2, _EB, _CW), jnp.float32),
            pltpu.VMEM((_EB, _CW), jnp.float32),
            pltpu.VMEM_SHARED((_ACC_ROWS, _CW), jnp.float32),
            pltpu.SemaphoreType.DMA((2,)),
            pltpu.SemaphoreType.DMA((2,)),
        ],
    )
    return f(*hs, *eas, srcs, dsts)


def kernel(h, edge_index, edge_attr, W_in, b_in, W_e, b_e, mlp_W0, mlp_b0,
           mlp_W1, mlp_b1, mlp_W2, mlp_b2, mlp_Wo, mlp_bo, W_out, b_out):
    srcs = edge_index[0].reshape(_NS, _NBLK, _EB)
    dsts = edge_index[1].reshape(_NS, _NBLK, _EB)
    hs = _prep_h(h, W_in, b_in)
    eas = _prep_ea(edge_attr, W_e, b_e)
    for i in range(L):
        aggrs = _edge_pass(hs, eas, srcs, dsts)
        hs = _mlp(hs, aggrs, mlp_W0[i], mlp_b0[i], mlp_W1[i], mlp_b1[i],
                  mlp_W2[i], mlp_b2[i], mlp_Wo[i], mlp_bo[i])
    return _proj_out(hs, W_out, b_out)


# speedup vs baseline: 2.3608x; 2.3608x over previous
import functools

import jax
import jax.numpy as jnp
from jax import lax
from jax.experimental import pallas as pl
from jax.experimental.pallas import tpu as pltpu
from jax.experimental.pallas import tpu_sc as plsc

N = 10000
E = 160000
D_IN = 256
H = 512
D_OUT = 256
D_EDGE = 16
L = 3

_ROW_BLK = 2000
_EDGE_BLK = 4000
_CW = 128
_NCHUNK = H // _CW

_NS = 16
_EPT = E // _NS
_EB = 80
_NBLK = _EPT // _EB
_NH = 5112
_TRASH = _NH
_ACC_ROWS = 5120
_NPAD = 2 * _NH
_STRIPE = _ACC_ROWS // _NS


def _prep_h_body(h_ref, w_ref, b_ref, o0, o1, o2, o3):
    y = h_ref[...] @ w_ref[...] + b_ref[...]
    for c, o in enumerate((o0, o1, o2, o3)):
        o[...] = y[:, c * _CW:(c + 1) * _CW]


def _prep_h(h, W_in, b_in):
    cspec = pl.BlockSpec((_ROW_BLK, _CW), lambda i: (i, 0))
    return pl.pallas_call(
        _prep_h_body,
        grid=(N // _ROW_BLK,),
        in_specs=[
            pl.BlockSpec((_ROW_BLK, D_IN), lambda i: (i, 0)),
            pl.BlockSpec((D_IN, H), lambda i: (0, 0)),
            pl.BlockSpec((1, H), lambda i: (0, 0)),
        ],
        out_specs=[cspec] * _NCHUNK,
        out_shape=[jax.ShapeDtypeStruct((N, _CW), jnp.float32)] * _NCHUNK,
    )(h, W_in, b_in.reshape(1, H))


def _prep_ea_body(ea_ref, w_ref, b_ref, o0, o1, o2, o3):
    y = ea_ref[...] @ w_ref[...] + b_ref[...]
    for c, o in enumerate((o0, o1, o2, o3)):
        o[...] = y[:, c * _CW:(c + 1) * _CW]


def _prep_ea(edge_attr, W_e, b_e):
    cspec = pl.BlockSpec((_EDGE_BLK, _CW), lambda i: (i, 0))
    return pl.pallas_call(
        _prep_ea_body,
        grid=(E // _EDGE_BLK,),
        in_specs=[
            pl.BlockSpec((_EDGE_BLK, D_EDGE), lambda i: (i, 0)),
            pl.BlockSpec((D_EDGE, H), lambda i: (0, 0)),
            pl.BlockSpec((1, H), lambda i: (0, 0)),
        ],
        out_specs=[cspec] * _NCHUNK,
        out_shape=[jax.ShapeDtypeStruct((E, _CW), jnp.float32)] * _NCHUNK,
    )(edge_attr, W_e, b_e.reshape(1, H))


def _mlp_body(h0, h1, h2, h3, a0, a1, a2, a3, w0_ref, b0_ref, w1_ref, b1_ref,
              w2_ref, b2_ref, wo_ref, bo_ref, o0, o1, o2, o3):
    x = jnp.concatenate(
        [hc[...] + ac[...] for hc, ac in zip((h0, h1, h2, h3),
                                             (a0, a1, a2, a3))], axis=1)
    x = jnp.maximum(x @ w0_ref[...] + b0_ref[...], 0.0)
    x = jnp.maximum(x @ w1_ref[...] + b1_ref[...], 0.0)
    x = jnp.maximum(x @ w2_ref[...] + b2_ref[...], 0.0)
    x = jnp.maximum(x @ wo_ref[...] + bo_ref[...], 0.0)
    for c, o in enumerate((o0, o1, o2, o3)):
        o[...] = x[:, c * _CW:(c + 1) * _CW]


def _mlp(hs, aggrs, W0, b0, W1, b1, W2, b2, Wo, bo):
    wspec = pl.BlockSpec((H, H), lambda i: (0, 0))
    bspec = pl.BlockSpec((1, H), lambda i: (0, 0))
    cspec = pl.BlockSpec((_ROW_BLK, _CW), lambda i: (i, 0))
    return pl.pallas_call(
        _mlp_body,
        grid=(N // _ROW_BLK,),
        in_specs=[cspec] * 8 + [wspec, bspec] * 4,
        out_specs=[cspec] * _NCHUNK,
        out_shape=[jax.ShapeDtypeStruct((N, _CW), jnp.float32)] * _NCHUNK,
    )(*hs, *aggrs, W0, b0.reshape(1, H), W1, b1.reshape(1, H), W2,
      b2.reshape(1, H), Wo, bo.reshape(1, H))


def _proj_out_body(h0, h1, h2, h3, w_ref, b_ref, o_ref):
    x = jnp.concatenate([hc[...] for hc in (h0, h1, h2, h3)], axis=1)
    o_ref[...] = x @ w_ref[...] + b_ref[...]


def _proj_out(hs, W_out, b_out):
    cspec = pl.BlockSpec((_ROW_BLK, _CW), lambda i: (i, 0))
    return pl.pallas_call(
        _proj_out_body,
        grid=(N // _ROW_BLK,),
        in_specs=[cspec] * _NCHUNK + [
            pl.BlockSpec((H, D_OUT), lambda i: (0, 0)),
            pl.BlockSpec((1, D_OUT), lambda i: (0, 0)),
        ],
        out_specs=pl.BlockSpec((_ROW_BLK, D_OUT), lambda i: (i, 0)),
        out_shape=jax.ShapeDtypeStruct((N, D_OUT), jnp.float32),
    )(*hs, W_out, b_out.reshape(1, D_OUT))


def _edge_body(ht0, ht1, ht2, ht3, ea0, ea1, ea2, ea3, srcs, dsts,
               o0, o1, o2, o3, idx_src, idx_dst, adj_stage, gbuf, ebuf, sbuf,
               accum, gsem, esem):
    sid = lax.axis_index("s")
    cid = lax.axis_index("c")
    base_row = sid * _STRIPE

    pltpu.sync_copy(srcs.at[sid], idx_src)
    pltpu.sync_copy(dsts.at[sid], idx_dst)

    zvec = jnp.zeros((16,), jnp.float32)

    def _zero_accum():
        gb = gbuf.at[0]

        def zrow(r, carry):
            for k in range(_CW // 16):
                gb[r, pl.ds(k * 16, 16)] = zvec
            return carry

        lax.fori_loop(0, _EB, zrow, 0, unroll=4)
        for i in range(_STRIPE // _EB):
            pltpu.sync_copy(gb, accum.at[pl.ds(base_row + i * _EB, _EB)])

    _trash_v = _TRASH + (lax.iota(jnp.int32, 16) & 7)

    def _make_adj(b, half):
        lo = half * _NH
        for k in range(_EB // 16):
            adj = idx_src[b, pl.ds(k * 16, 16)] - lo
            ok = (adj >= 0) & (adj < _NH)
            adj_stage[0, pl.ds(k * 16, 16)] = jnp.where(ok, adj, _trash_v)

    def _gather_descr(b, s, ht, ea):
        g = pltpu.make_async_copy(ht.at[idx_dst.at[b]], gbuf.at[s],
                                  gsem.at[s])
        e = pltpu.make_async_copy(
            ea.at[pl.ds(sid * _EPT + b * _EB, _EB)], ebuf.at[s], esem.at[s])
        return g, e

    def _compute(s):
        gb = gbuf.at[s]
        eb = ebuf.at[s]
        nk = _CW // 16

        def load_row(r):
            return ([gb[r, pl.ds(k * 16, 16)] for k in range(nk)],
                    [eb[r, pl.ds(k * 16, 16)] for k in range(nk)])

        def row(r, carry):
            gs, es = carry
            nxt = load_row(r + 1)
            for k in range(nk):
                sbuf[r, pl.ds(k * 16, 16)] = jnp.maximum(gs[k] + es[k], 0.0)
            return nxt

        gs, es = lax.fori_loop(0, _EB - 1, row, load_row(0), unroll=2)
        for k in range(nk):
            sbuf[_EB - 1, pl.ds(k * 16, 16)] = jnp.maximum(gs[k] + es[k], 0.0)

    def _run_pass(ht, ea, out, half):
        _zero_accum()
        plsc.subcore_barrier()

        for s in (0, 1):
            g, e = _gather_descr(s, s, ht, ea)
            g.start()
            e.start()

        def _do_block(b, s, start_next):
            g, e = _gather_descr(b, s, ht, ea)
            g.wait()
            e.wait()
            _compute(s)
            _make_adj(b, half)
            pltpu.sync_copy(sbuf, accum.at[adj_stage.at[0]], add=True)
            if start_next:
                @pl.when(b + 2 < _NBLK)
                def _():
                    gn, en = _gather_descr(b + 2, s, ht, ea)
                    gn.start()
                    en.start()

        def step(gi, carry):
            for s in (0, 1):
                _do_block(2 * gi + s, s, True)
            return carry

        lax.fori_loop(0, _NBLK // 2, step, 0)
        if _NBLK % 2:
            _do_block(_NBLK - 1, (_NBLK - 1) % 2, False)
        plsc.subcore_barrier()

        @pl.when(sid < _NS - 1)
        def _():
            pltpu.sync_copy(
                accum.at[pl.ds(base_row, _STRIPE)],
                out.at[pl.ds(half * _NH + base_row, _STRIPE)])

        @pl.when(sid == _NS - 1)
        def _():
            pltpu.sync_copy(
                accum.at[pl.ds(base_row, _NH - (_NS - 1) * _STRIPE)],
                out.at[pl.ds(half * _NH + base_row,
                             _NH - (_NS - 1) * _STRIPE)])

        plsc.subcore_barrier()

    @pl.when(cid == 0)
    def _():
        _run_pass(ht0, ea0, o0, 0)
        _run_pass(ht0, ea0, o0, 1)
        _run_pass(ht1, ea1, o1, 0)
        _run_pass(ht1, ea1, o1, 1)

    @pl.when(cid == 1)
    def _():
        _run_pass(ht2, ea2, o2, 0)
        _run_pass(ht2, ea2, o2, 1)
        _run_pass(ht3, ea3, o3, 0)
        _run_pass(ht3, ea3, o3, 1)


def _edge_pass(hs, eas, srcs, dsts):
    f = pl.kernel(
        _edge_body,
        out_type=[jax.ShapeDtypeStruct((_NPAD, _CW), jnp.float32)] * _NCHUNK,
        mesh=plsc.VectorSubcoreMesh(core_axis_name="c", subcore_axis_name="s"),
        scratch_types=[
            pltpu.VMEM((_NBLK, _EB), jnp.int32),
            pltpu.VMEM((_NBLK, _EB), jnp.int32),
            pltpu.VMEM((8, _EB), jnp.int32),
            pltpu.VMEM((2, _EB, _CW), jnp.float32),
            pltpu.VMEM((2, _EB, _CW), jnp.float32),
            pltpu.VMEM((_EB, _CW), jnp.float32),
            pltpu.VMEM_SHARED((_ACC_ROWS, _CW), jnp.float32),
            pltpu.SemaphoreType.DMA((2,)),
            pltpu.SemaphoreType.DMA((2,)),
        ],
    )
    return f(*hs, *eas, srcs, dsts)


def kernel(h, edge_index, edge_attr, W_in, b_in, W_e, b_e, mlp_W0, mlp_b0,
           mlp_W1, mlp_b1, mlp_W2, mlp_b2, mlp_Wo, mlp_bo, W_out, b_out):
    srcs = edge_index[0].reshape(_NS, _NBLK, _EB)
    dsts = edge_index[1].reshape(_NS, _NBLK, _EB)
    hs = _prep_h(h, W_in, b_in)
    eas = _prep_ea(edge_attr, W_e, b_e)
    for i in range(L):
        aggrs = _edge_pass(hs, eas, srcs, dsts)
        hs = _mlp(hs, aggrs, mlp_W0[i], mlp_b0[i], mlp_W1[i], mlp_b1[i],
                  mlp_W2[i], mlp_b2[i], mlp_Wo[i], mlp_bo[i])
    return _proj_out(hs, W_out, b_out)


# speedup vs baseline: 2.4937x; 1.0563x over previous
import functools

import jax
import jax.numpy as jnp
from jax import lax
from jax.experimental import pallas as pl
from jax.experimental.pallas import tpu as pltpu
from jax.experimental.pallas import tpu_sc as plsc

N = 10000
E = 160000
D_IN = 256
H = 512
D_OUT = 256
D_EDGE = 16
L = 3

_ROW_BLK = 2000
_EDGE_BLK = 4128
_CW = 128
_NCHUNK = H // _CW

_NS = 16
_EPT = E // _NS
_EB = 80
_NBLK = _EPT // _EB
_NH = 5112
_EPAD = 165120
_TRASH = _NH
_ACC_ROWS = 5120
_NPAD = 2 * _NH
_STRIPE = _ACC_ROWS // _NS


def _prep_h_body(h_ref, w_ref, b_ref, o0, o1, o2, o3):
    y = h_ref[...] @ w_ref[...] + b_ref[...]
    for c, o in enumerate((o0, o1, o2, o3)):
        o[...] = y[:, c * _CW:(c + 1) * _CW]


def _prep_h(h, W_in, b_in):
    cspec = pl.BlockSpec((_ROW_BLK, _CW), lambda i: (i, 0))
    return pl.pallas_call(
        _prep_h_body,
        grid=(N // _ROW_BLK,),
        in_specs=[
            pl.BlockSpec((_ROW_BLK, D_IN), lambda i: (i, 0)),
            pl.BlockSpec((D_IN, H), lambda i: (0, 0)),
            pl.BlockSpec((1, H), lambda i: (0, 0)),
        ],
        out_specs=[cspec] * _NCHUNK,
        out_shape=[jax.ShapeDtypeStruct((N, _CW), jnp.float32)] * _NCHUNK,
    )(h, W_in, b_in.reshape(1, H))


def _prep_ea_body(ea_ref, w_ref, b_ref, o0, o1, o2, o3):
    y = ea_ref[...] @ w_ref[...] + b_ref[...]
    for c, o in enumerate((o0, o1, o2, o3)):
        o[...] = y[:, c * _CW:(c + 1) * _CW]


def _prep_ea(edge_attr, W_e, b_e):
    cspec = pl.BlockSpec((_EDGE_BLK, _CW), lambda i: (i, 0))
    return pl.pallas_call(
        _prep_ea_body,
        grid=(_EPAD // _EDGE_BLK,),
        in_specs=[
            pl.BlockSpec((_EDGE_BLK, D_EDGE), lambda i: (i, 0)),
            pl.BlockSpec((D_EDGE, H), lambda i: (0, 0)),
            pl.BlockSpec((1, H), lambda i: (0, 0)),
        ],
        out_specs=[cspec] * _NCHUNK,
        out_shape=[jax.ShapeDtypeStruct((_EPAD, _CW), jnp.float32)] * _NCHUNK,
    )(edge_attr, W_e, b_e.reshape(1, H))


def _mlp_body(h0, h1, h2, h3, a0, a1, a2, a3, w0_ref, b0_ref, w1_ref, b1_ref,
              w2_ref, b2_ref, wo_ref, bo_ref, o0, o1, o2, o3):
    x = jnp.concatenate(
        [hc[...] + ac[...] for hc, ac in zip((h0, h1, h2, h3),
                                             (a0, a1, a2, a3))], axis=1)
    x = jnp.maximum(x @ w0_ref[...] + b0_ref[...], 0.0)
    x = jnp.maximum(x @ w1_ref[...] + b1_ref[...], 0.0)
    x = jnp.maximum(x @ w2_ref[...] + b2_ref[...], 0.0)
    x = jnp.maximum(x @ wo_ref[...] + bo_ref[...], 0.0)
    for c, o in enumerate((o0, o1, o2, o3)):
        o[...] = x[:, c * _CW:(c + 1) * _CW]


def _mlp(hs, aggrs, W0, b0, W1, b1, W2, b2, Wo, bo):
    wspec = pl.BlockSpec((H, H), lambda i: (0, 0))
    bspec = pl.BlockSpec((1, H), lambda i: (0, 0))
    cspec = pl.BlockSpec((_ROW_BLK, _CW), lambda i: (i, 0))
    return pl.pallas_call(
        _mlp_body,
        grid=(N // _ROW_BLK,),
        in_specs=[cspec] * 8 + [wspec, bspec] * 4,
        out_specs=[cspec] * _NCHUNK,
        out_shape=[jax.ShapeDtypeStruct((N, _CW), jnp.float32)] * _NCHUNK,
    )(*hs, *aggrs, W0, b0.reshape(1, H), W1, b1.reshape(1, H), W2,
      b2.reshape(1, H), Wo, bo.reshape(1, H))


def _proj_out_body(h0, h1, h2, h3, w_ref, b_ref, o_ref):
    x = jnp.concatenate([hc[...] for hc in (h0, h1, h2, h3)], axis=1)
    o_ref[...] = x @ w_ref[...] + b_ref[...]


def _proj_out(hs, W_out, b_out):
    cspec = pl.BlockSpec((_ROW_BLK, _CW), lambda i: (i, 0))
    return pl.pallas_call(
        _proj_out_body,
        grid=(N // _ROW_BLK,),
        in_specs=[cspec] * _NCHUNK + [
            pl.BlockSpec((H, D_OUT), lambda i: (0, 0)),
            pl.BlockSpec((1, D_OUT), lambda i: (0, 0)),
        ],
        out_specs=pl.BlockSpec((_ROW_BLK, D_OUT), lambda i: (i, 0)),
        out_shape=jax.ShapeDtypeStruct((N, D_OUT), jnp.float32),
    )(*hs, W_out, b_out.reshape(1, D_OUT))


_TPR = _EPT + 320
_NBT = _TPR // _EB
_MAXP = (_NBT + 1) // 2


def _edge_body(ht0, ht1, ht2, ht3, ea0, ea1, ea2, ea3, dst_hbm, adj_hbm,
               cnt_hbm, o0, o1, o2, o3, dstbuf, adjbuf, cntbuf, gbuf, ebuf,
               sbuf, accum, gsem, esem):
    sid = lax.axis_index("s")
    cid = lax.axis_index("c")
    base_row = sid * _STRIPE

    pltpu.sync_copy(dst_hbm.at[sid], dstbuf)
    pltpu.sync_copy(adj_hbm.at[sid], adjbuf)
    pltpu.sync_copy(cnt_hbm.at[sid], cntbuf)

    cntv = cntbuf[pl.ds(0, 16)]
    np_a = cntv[0]
    np_b = cntv[1]

    zvec = jnp.zeros((16,), jnp.float32)

    def _zero_accum():
        gb = gbuf.at[0]

        def zrow(r, carry):
            for k in range(_CW // 16):
                gb[r, pl.ds(k * 16, 16)] = zvec
            return carry

        lax.fori_loop(0, _EB, zrow, 0, unroll=4)
        for i in range(_STRIPE // _EB):
            pltpu.sync_copy(gb, accum.at[pl.ds(base_row + i * _EB, _EB)])

    def _descr(row, s, ht, ea):
        g = pltpu.make_async_copy(ht.at[dstbuf.at[row]], gbuf.at[s],
                                  gsem.at[s])
        e = pltpu.make_async_copy(
            ea.at[pl.ds((sid * _NBT + row) * _EB, _EB)], ebuf.at[s],
            esem.at[s])
        return g, e

    def _compute(s):
        gb = gbuf.at[s]
        eb = ebuf.at[s]
        nk = _CW // 16

        def load_row(r):
            return ([gb[r, pl.ds(k * 16, 16)] for k in range(nk)],
                    [eb[r, pl.ds(k * 16, 16)] for k in range(nk)])

        def row(r, carry):
            gs, es = carry
            nxt = load_row(r + 1)
            for k in range(nk):
                sbuf[r, pl.ds(k * 16, 16)] = jnp.maximum(gs[k] + es[k], 0.0)
            return nxt

        gs, es = lax.fori_loop(0, _EB - 1, row, load_row(0), unroll=2)
        for k in range(nk):
            sbuf[_EB - 1, pl.ds(k * 16, 16)] = jnp.maximum(gs[k] + es[k], 0.0)

    def _run_pass(ht, ea, out, rowbase, nblocks, half):
        _zero_accum()
        plsc.subcore_barrier()

        for s in (0, 1):
            g, e = _descr(rowbase + s, s, ht, ea)
            g.start()
            e.start()

        def _do_block(b, s):
            @pl.when(b < nblocks)
            def _():
                row = rowbase + b
                g, e = _descr(row, s, ht, ea)
                g.wait()
                e.wait()
                _compute(s)
                pltpu.sync_copy(sbuf, accum.at[adjbuf.at[row]], add=True)

                @pl.when(b + 2 < nblocks)
                def _():
                    gn, en = _descr(row + 2, s, ht, ea)
                    gn.start()
                    en.start()

        def step(gi, carry):
            _do_block(2 * gi, 0)
            _do_block(2 * gi + 1, 1)
            return carry

        lax.fori_loop(0, _MAXP, step, 0)
        plsc.subcore_barrier()

        @pl.when(sid < _NS - 1)
        def _():
            pltpu.sync_copy(
                accum.at[pl.ds(base_row, _STRIPE)],
                out.at[pl.ds(half * _NH + base_row, _STRIPE)])

        @pl.when(sid == _NS - 1)
        def _():
            pltpu.sync_copy(
                accum.at[pl.ds(base_row, _NH - (_NS - 1) * _STRIPE)],
                out.at[pl.ds(half * _NH + base_row,
                             _NH - (_NS - 1) * _STRIPE)])

        plsc.subcore_barrier()

    def _chunk(ht, ea, out):
        _run_pass(ht, ea, out, 0, 2 * np_a, 0)
        _run_pass(ht, ea, out, 2 * np_a, 2 * np_b, 1)

    @pl.when(cid == 0)
    def _():
        _chunk(ht0, ea0, o0)
        _chunk(ht1, ea1, o1)

    @pl.when(cid == 1)
    def _():
        _chunk(ht2, ea2, o2)
        _chunk(ht3, ea3, o3)


def _edge_pass(hs, eas, dst_pad, adj_pad, cnts):
    f = pl.kernel(
        _edge_body,
        out_type=[jax.ShapeDtypeStruct((_NPAD, _CW), jnp.float32)] * _NCHUNK,
        mesh=plsc.VectorSubcoreMesh(core_axis_name="c", subcore_axis_name="s"),
        scratch_types=[
            pltpu.VMEM((_NBT, _EB), jnp.int32),
            pltpu.VMEM((_NBT, _EB), jnp.int32),
            pltpu.VMEM((16,), jnp.int32),
            pltpu.VMEM((2, _EB, _CW), jnp.float32),
            pltpu.VMEM((2, _EB, _CW), jnp.float32),
            pltpu.VMEM((_EB, _CW), jnp.float32),
            pltpu.VMEM_SHARED((_ACC_ROWS, _CW), jnp.float32),
            pltpu.SemaphoreType.DMA((2,)),
            pltpu.SemaphoreType.DMA((2,)),
        ],
    )
    return f(*hs, *eas, dst_pad, adj_pad, cnts)


def _partition_edges(edge_index, edge_attr):
    srcs2 = edge_index[0].reshape(_NS, _EPT)
    dsts2 = edge_index[1].reshape(_NS, _EPT)
    is_b = srcs2 >= _NH
    order = jnp.argsort(is_b, axis=1, stable=True)
    n_a = (_EPT - jnp.sum(is_b, axis=1)).astype(jnp.int32)
    np_a = jnp.maximum((n_a + 159) // 160, 1)
    np_b = jnp.maximum(((_EPT - n_a) + 159) // 160, 1)
    sec_a = np_a * 160
    n_b = _EPT - n_a

    k = jnp.arange(_TPR, dtype=jnp.int32)[None, :]
    n_a_ = n_a[:, None]
    in_a = k < n_a_
    in_b = (k >= sec_a[:, None]) & (k < (sec_a + n_b)[:, None])
    valid = in_a | in_b
    jj = jnp.where(in_a, k, n_a_ + (k - sec_a[:, None]))
    jj = jnp.clip(jj, 0, _EPT - 1)
    o = jnp.take_along_axis(order, jj, axis=1)
    dstv = jnp.take_along_axis(dsts2, o, axis=1)
    srcv = jnp.take_along_axis(srcs2, o, axis=1)
    dst_pad = jnp.where(valid, dstv, 0).reshape(_NS, _NBT, _EB)
    adj = srcv - jnp.where(srcv >= _NH, _NH, 0)
    adj_pad = jnp.where(valid, adj, _TRASH + (k & 7)).reshape(_NS, _NBT, _EB)
    eav = jnp.take_along_axis(edge_attr.reshape(_NS, _EPT, D_EDGE),
                              o[:, :, None], axis=1)
    ea_pad = jnp.where(valid[:, :, None], eav, 0.0).reshape(_NS * _TPR,
                                                            D_EDGE)
    cnts = jnp.zeros((_NS, 16), jnp.int32)
    cnts = cnts.at[:, 0].set(np_a).at[:, 1].set(np_b)
    return dst_pad, adj_pad, ea_pad, cnts


def kernel(h, edge_index, edge_attr, W_in, b_in, W_e, b_e, mlp_W0, mlp_b0,
           mlp_W1, mlp_b1, mlp_W2, mlp_b2, mlp_Wo, mlp_bo, W_out, b_out):
    dst_pad, adj_pad, ea_pad, cnts = _partition_edges(edge_index, edge_attr)
    hs = _prep_h(h, W_in, b_in)
    eas = _prep_ea(ea_pad, W_e, b_e)
    for i in range(L):
        aggrs = _edge_pass(hs, eas, dst_pad, adj_pad, cnts)
        hs = _mlp(hs, aggrs, mlp_W0[i], mlp_b0[i], mlp_W1[i], mlp_b1[i],
                  mlp_W2[i], mlp_b2[i], mlp_Wo[i], mlp_bo[i])
    return _proj_out(hs, W_out, b_out)


# speedup vs baseline: 2.5192x; 1.0102x over previous
import functools

import jax
import jax.numpy as jnp
from jax import lax
from jax.experimental import pallas as pl
from jax.experimental.pallas import tpu as pltpu
from jax.experimental.pallas import tpu_sc as plsc

N = 10000
E = 160000
D_IN = 256
H = 512
D_OUT = 256
D_EDGE = 16
L = 3

_ROW_BLK = 2000
_EDGE_BLK = 4128
_CW = 128
_NCHUNK = H // _CW

_NS = 16
_EPT = E // _NS
_EB = 80
_NBLK = _EPT // _EB
_NH = 5112
_EPAD = 165120
_TRASH = _NH
_ACC_ROWS = 5120
_NPAD = 2 * _NH
_STRIPE = _ACC_ROWS // _NS


def _prep_h_body(h_ref, w_ref, b_ref, o0, o1, o2, o3):
    y = h_ref[...] @ w_ref[...] + b_ref[...]
    for c, o in enumerate((o0, o1, o2, o3)):
        o[...] = y[:, c * _CW:(c + 1) * _CW]


def _prep_h(h, W_in, b_in):
    cspec = pl.BlockSpec((_ROW_BLK, _CW), lambda i: (i, 0))
    return pl.pallas_call(
        _prep_h_body,
        grid=(N // _ROW_BLK,),
        in_specs=[
            pl.BlockSpec((_ROW_BLK, D_IN), lambda i: (i, 0)),
            pl.BlockSpec((D_IN, H), lambda i: (0, 0)),
            pl.BlockSpec((1, H), lambda i: (0, 0)),
        ],
        out_specs=[cspec] * _NCHUNK,
        out_shape=[jax.ShapeDtypeStruct((N, _CW), jnp.float32)] * _NCHUNK,
    )(h, W_in, b_in.reshape(1, H))


def _prep_ea_body(ea_ref, w_ref, b_ref, o0, o1, o2, o3):
    y = ea_ref[...] @ w_ref[...] + b_ref[...]
    for c, o in enumerate((o0, o1, o2, o3)):
        o[...] = y[:, c * _CW:(c + 1) * _CW]


def _prep_ea(edge_attr, W_e, b_e):
    cspec = pl.BlockSpec((_EDGE_BLK, _CW), lambda i: (i, 0))
    return pl.pallas_call(
        _prep_ea_body,
        grid=(_EPAD // _EDGE_BLK,),
        in_specs=[
            pl.BlockSpec((_EDGE_BLK, D_EDGE), lambda i: (i, 0)),
            pl.BlockSpec((D_EDGE, H), lambda i: (0, 0)),
            pl.BlockSpec((1, H), lambda i: (0, 0)),
        ],
        out_specs=[cspec] * _NCHUNK,
        out_shape=[jax.ShapeDtypeStruct((_EPAD, _CW), jnp.float32)] * _NCHUNK,
    )(edge_attr, W_e, b_e.reshape(1, H))


def _mlp_body(h0, h1, h2, h3, a0, a1, a2, a3, w0_ref, b0_ref, w1_ref, b1_ref,
              w2_ref, b2_ref, wo_ref, bo_ref, o0, o1, o2, o3):
    x = jnp.concatenate(
        [hc[...] + ac[...] for hc, ac in zip((h0, h1, h2, h3),
                                             (a0, a1, a2, a3))], axis=1)
    x = jnp.maximum(x @ w0_ref[...] + b0_ref[...], 0.0)
    x = jnp.maximum(x @ w1_ref[...] + b1_ref[...], 0.0)
    x = jnp.maximum(x @ w2_ref[...] + b2_ref[...], 0.0)
    x = jnp.maximum(x @ wo_ref[...] + bo_ref[...], 0.0)
    for c, o in enumerate((o0, o1, o2, o3)):
        o[...] = x[:, c * _CW:(c + 1) * _CW]


def _mlp(hs, aggrs, W0, b0, W1, b1, W2, b2, Wo, bo):
    wspec = pl.BlockSpec((H, H), lambda i: (0, 0))
    bspec = pl.BlockSpec((1, H), lambda i: (0, 0))
    cspec = pl.BlockSpec((_ROW_BLK, _CW), lambda i: (i, 0))
    return pl.pallas_call(
        _mlp_body,
        grid=(N // _ROW_BLK,),
        in_specs=[cspec] * 8 + [wspec, bspec] * 4,
        out_specs=[cspec] * _NCHUNK,
        out_shape=[jax.ShapeDtypeStruct((N, _CW), jnp.float32)] * _NCHUNK,
    )(*hs, *aggrs, W0, b0.reshape(1, H), W1, b1.reshape(1, H), W2,
      b2.reshape(1, H), Wo, bo.reshape(1, H))


def _proj_out_body(h0, h1, h2, h3, w_ref, b_ref, o_ref):
    x = jnp.concatenate([hc[...] for hc in (h0, h1, h2, h3)], axis=1)
    o_ref[...] = x @ w_ref[...] + b_ref[...]


def _proj_out(hs, W_out, b_out):
    cspec = pl.BlockSpec((_ROW_BLK, _CW), lambda i: (i, 0))
    return pl.pallas_call(
        _proj_out_body,
        grid=(N // _ROW_BLK,),
        in_specs=[cspec] * _NCHUNK + [
            pl.BlockSpec((H, D_OUT), lambda i: (0, 0)),
            pl.BlockSpec((1, D_OUT), lambda i: (0, 0)),
        ],
        out_specs=pl.BlockSpec((_ROW_BLK, D_OUT), lambda i: (i, 0)),
        out_shape=jax.ShapeDtypeStruct((N, D_OUT), jnp.float32),
    )(*hs, W_out, b_out.reshape(1, D_OUT))


_TPR = _EPT + 320
_NBT = _TPR // _EB
_MAXP = (_NBT + 1) // 2


def _edge_body(ht0, ht1, ht2, ht3, ea0, ea1, ea2, ea3, dst_hbm, adj_hbm,
               cnt_hbm, o0, o1, o2, o3, dstbuf, adjbuf, cntbuf, gbuf, ebuf,
               sbuf, accum, gsem, esem, ssem):
    sid = lax.axis_index("s")
    cid = lax.axis_index("c")
    base_row = sid * _STRIPE

    pltpu.sync_copy(dst_hbm.at[sid], dstbuf)
    pltpu.sync_copy(adj_hbm.at[sid], adjbuf)
    pltpu.sync_copy(cnt_hbm.at[sid], cntbuf)

    cntv = cntbuf[pl.ds(0, 16)]
    np_a = cntv[0]
    np_b = cntv[1]

    zvec = jnp.zeros((16,), jnp.float32)

    def _zero_accum():
        gb = gbuf.at[0]

        def zrow(r, carry):
            for k in range(_CW // 16):
                gb[r, pl.ds(k * 16, 16)] = zvec
            return carry

        lax.fori_loop(0, _EB, zrow, 0, unroll=4)
        for i in range(_STRIPE // _EB):
            pltpu.sync_copy(gb, accum.at[pl.ds(base_row + i * _EB, _EB)])

    def _descr(row, s, ht, ea):
        g = pltpu.make_async_copy(ht.at[dstbuf.at[row]], gbuf.at[s],
                                  gsem.at[s])
        e = pltpu.make_async_copy(
            ea.at[pl.ds((sid * _NBT + row) * _EB, _EB)], ebuf.at[s],
            esem.at[s])
        return g, e

    def _compute(s):
        gb = gbuf.at[s]
        eb = ebuf.at[s]
        sb = sbuf
        nk = _CW // 16

        def load_row(r):
            return ([gb[r, pl.ds(k * 16, 16)] for k in range(nk)],
                    [eb[r, pl.ds(k * 16, 16)] for k in range(nk)])

        def row(r, carry):
            gs, es = carry
            nxt = load_row(r + 1)
            for k in range(nk):
                sb[r, pl.ds(k * 16, 16)] = jnp.maximum(gs[k] + es[k], 0.0)
            return nxt

        gs, es = lax.fori_loop(0, _EB - 1, row, load_row(0), unroll=2)
        for k in range(nk):
            sb[_EB - 1, pl.ds(k * 16, 16)] = jnp.maximum(gs[k] + es[k], 0.0)

    def _run_pass(ht, ea, out, rowbase, nblocks, half):
        _zero_accum()
        plsc.subcore_barrier()

        for s in (0, 1):
            g, e = _descr(rowbase + s, s, ht, ea)
            g.start()
            e.start()

        def _scat_wait(row):
            pltpu.make_async_copy(sbuf, accum.at[adjbuf.at[row]],
                                  ssem.at[0]).wait()

        def _scat_start(row):
            pltpu.async_copy(sbuf, accum.at[adjbuf.at[row]],
                             ssem.at[0], add=True)

        def _do_block(b, s):
            @pl.when(b < nblocks)
            def _():
                row = rowbase + b
                g, e = _descr(row, s, ht, ea)
                g.wait()
                e.wait()

                @pl.when(b >= 1)
                def _():
                    _scat_wait(row)

                _compute(s)
                _scat_start(row)

                @pl.when(b + 2 < nblocks)
                def _():
                    gn, en = _descr(row + 2, s, ht, ea)
                    gn.start()
                    en.start()

        def step(gi, carry):
            _do_block(2 * gi, 0)
            _do_block(2 * gi + 1, 1)
            return carry

        lax.fori_loop(0, _MAXP, step, 0)
        _scat_wait(rowbase)
        plsc.subcore_barrier()

        @pl.when(sid < _NS - 1)
        def _():
            pltpu.sync_copy(
                accum.at[pl.ds(base_row, _STRIPE)],
                out.at[pl.ds(half * _NH + base_row, _STRIPE)])

        @pl.when(sid == _NS - 1)
        def _():
            pltpu.sync_copy(
                accum.at[pl.ds(base_row, _NH - (_NS - 1) * _STRIPE)],
                out.at[pl.ds(half * _NH + base_row,
                             _NH - (_NS - 1) * _STRIPE)])

        plsc.subcore_barrier()

    def _chunk(ht, ea, out):
        _run_pass(ht, ea, out, 0, 2 * np_a, 0)
        _run_pass(ht, ea, out, 2 * np_a, 2 * np_b, 1)

    @pl.when(cid == 0)
    def _():
        _chunk(ht0, ea0, o0)
        _chunk(ht1, ea1, o1)

    @pl.when(cid == 1)
    def _():
        _chunk(ht2, ea2, o2)
        _chunk(ht3, ea3, o3)


def _edge_pass(hs, eas, dst_pad, adj_pad, cnts):
    f = pl.kernel(
        _edge_body,
        out_type=[jax.ShapeDtypeStruct((_NPAD, _CW), jnp.float32)] * _NCHUNK,
        mesh=plsc.VectorSubcoreMesh(core_axis_name="c", subcore_axis_name="s"),
        scratch_types=[
            pltpu.VMEM((_NBT, _EB), jnp.int32),
            pltpu.VMEM((_NBT, _EB), jnp.int32),
            pltpu.VMEM((16,), jnp.int32),
            pltpu.VMEM((2, _EB, _CW), jnp.float32),
            pltpu.VMEM((2, _EB, _CW), jnp.float32),
            pltpu.VMEM((_EB, _CW), jnp.float32),
            pltpu.VMEM_SHARED((_ACC_ROWS, _CW), jnp.float32),
            pltpu.SemaphoreType.DMA((2,)),
            pltpu.SemaphoreType.DMA((2,)),
            pltpu.SemaphoreType.DMA((2,)),
        ],
    )
    return f(*hs, *eas, dst_pad, adj_pad, cnts)


def _partition_edges(edge_index, edge_attr):
    srcs2 = edge_index[0].reshape(_NS, _EPT)
    dsts2 = edge_index[1].reshape(_NS, _EPT)
    is_b = srcs2 >= _NH
    order = jnp.argsort(is_b, axis=1, stable=True)
    n_a = (_EPT - jnp.sum(is_b, axis=1)).astype(jnp.int32)
    np_a = jnp.maximum((n_a + 159) // 160, 1)
    np_b = jnp.maximum(((_EPT - n_a) + 159) // 160, 1)
    sec_a = np_a * 160
    n_b = _EPT - n_a

    k = jnp.arange(_TPR, dtype=jnp.int32)[None, :]
    n_a_ = n_a[:, None]
    in_a = k < n_a_
    in_b = (k >= sec_a[:, None]) & (k < (sec_a + n_b)[:, None])
    valid = in_a | in_b
    jj = jnp.where(in_a, k, n_a_ + (k - sec_a[:, None]))
    jj = jnp.clip(jj, 0, _EPT - 1)
    o = jnp.take_along_axis(order, jj, axis=1)
    dstv = jnp.take_along_axis(dsts2, o, axis=1)
    srcv = jnp.take_along_axis(srcs2, o, axis=1)
    dst_pad = jnp.where(valid, dstv, 0).reshape(_NS, _NBT, _EB)
    adj = srcv - jnp.where(srcv >= _NH, _NH, 0)
    adj_pad = jnp.where(valid, adj, _TRASH + (k & 7)).reshape(_NS, _NBT, _EB)
    eav = jnp.take_along_axis(edge_attr.reshape(_NS, _EPT, D_EDGE),
                              o[:, :, None], axis=1)
    ea_pad = jnp.where(valid[:, :, None], eav, 0.0).reshape(_NS * _TPR,
                                                            D_EDGE)
    cnts = jnp.zeros((_NS, 16), jnp.int32)
    cnts = cnts.at[:, 0].set(np_a).at[:, 1].set(np_b)
    return dst_pad, adj_pad, ea_pad, cnts


def kernel(h, edge_index, edge_attr, W_in, b_in, W_e, b_e, mlp_W0, mlp_b0,
           mlp_W1, mlp_b1, mlp_W2, mlp_b2, mlp_Wo, mlp_bo, W_out, b_out):
    dst_pad, adj_pad, ea_pad, cnts = _partition_edges(edge_index, edge_attr)
    hs = _prep_h(h, W_in, b_in)
    eas = _prep_ea(ea_pad, W_e, b_e)
    for i in range(L):
        aggrs = _edge_pass(hs, eas, dst_pad, adj_pad, cnts)
        hs = _mlp(hs, aggrs, mlp_W0[i], mlp_b0[i], mlp_W1[i], mlp_b1[i],
                  mlp_W2[i], mlp_b2[i], mlp_Wo[i], mlp_bo[i])
    return _proj_out(hs, W_out, b_out)
